# baseline (speedup 1.0000x reference)
import functools

import jax
import jax.numpy as jnp
from jax import lax
from jax.experimental import pallas as pl
from jax.experimental.pallas import tpu as pltpu
from jax.experimental.pallas import tpu_sc as plsc

_D_ID = 32
_D_T = 32
_D_OUT = _D_ID + _D_T


def _table_body(id_ref, t_ref, w1_ref, b1_ref, w2t_ref, b2_ref, out_ref):
    t = t_ref[...]
    t_scale = jnp.max(t) + 1e-6
    ta = t / t_scale
    h = jnp.maximum(ta * w1_ref[...] + b1_ref[...], 0.0)
    e_t = jnp.dot(h, w2t_ref[...], preferred_element_type=jnp.float32)
    e_t = e_t + b2_ref[...]
    out_ref[:, :_D_ID] = id_ref[...]
    out_ref[:, _D_ID:] = e_t


def _build_table(id_emb, t_from_A, W1, b1, W2, b2):
    n = id_emb.shape[0]
    return pl.pallas_call(
        _table_body,
        out_shape=jax.ShapeDtypeStruct((n, _D_OUT), jnp.float32),
    )(
        id_emb,
        t_from_A.reshape(n, 1),
        W1.reshape(1, _D_T),
        b1.reshape(1, _D_T),
        W2.T,
        b2.reshape(1, _D_T),
    )


_NC = 2
_NS = 16
_NW = _NC * _NS
_CHUNK = 512


def _gather_body(n_chunks, table_hbm, idx_hbm, out_hbm, idx_v, rows_v,
                 gsems, wsems):
    wid = lax.axis_index("s") * _NC + lax.axis_index("c")
    per_w = n_chunks * _CHUNK
    base = wid * per_w

    pltpu.sync_copy(idx_hbm.at[pl.ds(base, per_w)], idx_v)

    def idx_slice(m):
        return idx_v.at[pl.ds(m * _CHUNK, _CHUNK)]

    def start_gather(m, b):
        pltpu.async_copy(table_hbm.at[idx_slice(m)], rows_v.at[b], gsems.at[b])

    def wait_gather(m, b):
        pltpu.make_async_copy(
            table_hbm.at[idx_slice(m)], rows_v.at[b], gsems.at[b]).wait()

    def start_wb(m, b):
        pltpu.async_copy(
            rows_v.at[b], out_hbm.at[pl.ds(base + m * _CHUNK, _CHUNK)],
            wsems.at[b])

    def wait_wb(m, b):
        pltpu.make_async_copy(
            rows_v.at[b], out_hbm.at[pl.ds(base + m * _CHUNK, _CHUNK)],
            wsems.at[b]).wait()

    start_gather(0, 0)

    @pl.loop(0, n_chunks, step=2)
    def _pair(c):
        for b in range(2):
            m = c + b
            nb = 1 - b

            def _prefetch(m=m, nb=nb):
                @pl.when(m >= 1)
                def _():
                    wait_wb(m - 1, nb)
                start_gather(m + 1, nb)

            if b == 0:
                _prefetch()
            else:
                pl.when(m + 1 < n_chunks)(_prefetch)
            wait_gather(m, b)
            start_wb(m, b)

    wait_wb(n_chunks - 2, (n_chunks - 2) % 2)
    wait_wb(n_chunks - 1, (n_chunks - 1) % 2)


def _gather(table, idx_flat):
    n_idx = idx_flat.shape[0]
    assert n_idx % (_NW * _CHUNK) == 0
    n_chunks = n_idx // (_NW * _CHUNK)
    assert n_chunks % 2 == 0
    mesh = plsc.VectorSubcoreMesh(core_axis_name="c", subcore_axis_name="s")
    k = pl.kernel(
        functools.partial(_gather_body, n_chunks),
        out_type=jax.ShapeDtypeStruct((n_idx, _D_OUT), jnp.float32),
        mesh=mesh,
        scratch_types=[
            pltpu.VMEM((n_chunks * _CHUNK,), jnp.int32),
            pltpu.VMEM((2, _CHUNK, _D_OUT), jnp.float32),
            pltpu.SemaphoreType.DMA((2,)),
            pltpu.SemaphoreType.DMA((2,)),
        ],
        compiler_params=pltpu.CompilerParams(use_tc_tiling_on_sc=False),
    )
    return k(table, idx_flat)


@jax.jit
def kernel(station_ids, id_emb, t_from_A, W1, b1, W2, b2):
    B, L = station_ids.shape
    table = _build_table(id_emb, t_from_A, W1, b1, W2, b2)
    out = _gather(table, station_ids.reshape(B * L))
    return out.reshape(B, L, _D_OUT)


# speedup vs baseline: 1.3972x; 1.3972x over previous
import functools

import jax
import jax.numpy as jnp
from jax import lax
from jax.experimental import pallas as pl
from jax.experimental.pallas import tpu as pltpu
from jax.experimental.pallas import tpu_sc as plsc

_D_ID = 32
_D_T = 32
_D_OUT = _D_ID + _D_T


def _table_body(id_ref, t_ref, w1_ref, b1_ref, w2t_ref, b2_ref, out_ref):
    t = t_ref[...]
    t_scale = jnp.max(t) + 1e-6
    ta = t / t_scale
    h = jnp.maximum(ta * w1_ref[...] + b1_ref[...], 0.0)
    e_t = jnp.dot(h, w2t_ref[...], preferred_element_type=jnp.float32)
    e_t = e_t + b2_ref[...]
    out_ref[:, :_D_ID] = id_ref[...]
    out_ref[:, _D_ID:] = e_t


def _build_table(id_emb, t_from_A, W1, b1, W2, b2):
    n = id_emb.shape[0]
    return pl.pallas_call(
        _table_body,
        out_shape=jax.ShapeDtypeStruct((n, _D_OUT), jnp.float32),
    )(
        id_emb,
        t_from_A.reshape(n, 1),
        W1.reshape(1, _D_T),
        b1.reshape(1, _D_T),
        W2.T,
        b2.reshape(1, _D_T),
    )


_NC = 2
_NS = 16
_NW = _NC * _NS
_CHUNK = 512


def _gather_body(n_chunks, table_hbm, idx_hbm, out_hbm, idx_v, rows_v,
                 table_sh, gsems, wsems):
    wid = lax.axis_index("s") * _NC + lax.axis_index("c")
    per_w = n_chunks * _CHUNK
    base = wid * per_w

    @pl.when(lax.axis_index("s") == 0)
    def _():
        pltpu.sync_copy(table_hbm, table_sh)

    pltpu.sync_copy(idx_hbm.at[pl.ds(base, per_w)], idx_v)
    plsc.subcore_barrier()

    def idx_slice(m):
        return idx_v.at[pl.ds(m * _CHUNK, _CHUNK)]

    def start_gather(m, b):
        pltpu.async_copy(table_sh.at[idx_slice(m)], rows_v.at[b], gsems.at[b])

    def wait_gather(m, b):
        pltpu.make_async_copy(
            table_sh.at[idx_slice(m)], rows_v.at[b], gsems.at[b]).wait()

    def start_wb(m, b):
        pltpu.async_copy(
            rows_v.at[b], out_hbm.at[pl.ds(base + m * _CHUNK, _CHUNK)],
            wsems.at[b])

    def wait_wb(m, b):
        pltpu.make_async_copy(
            rows_v.at[b], out_hbm.at[pl.ds(base + m * _CHUNK, _CHUNK)],
            wsems.at[b]).wait()

    start_gather(0, 0)

    @pl.loop(0, n_chunks, step=2)
    def _pair(c):
        for b in range(2):
            m = c + b
            nb = 1 - b

            def _prefetch(m=m, nb=nb):
                @pl.when(m >= 1)
                def _():
                    wait_wb(m - 1, nb)
                start_gather(m + 1, nb)

            if b == 0:
                _prefetch()
            else:
                pl.when(m + 1 < n_chunks)(_prefetch)
            wait_gather(m, b)
            start_wb(m, b)

    wait_wb(n_chunks - 2, (n_chunks - 2) % 2)
    wait_wb(n_chunks - 1, (n_chunks - 1) % 2)


def _gather(table, idx_flat):
    n_idx = idx_flat.shape[0]
    assert n_idx % (_NW * _CHUNK) == 0
    n_chunks = n_idx // (_NW * _CHUNK)
    assert n_chunks % 2 == 0
    mesh = plsc.VectorSubcoreMesh(core_axis_name="c", subcore_axis_name="s")
    k = pl.kernel(
        functools.partial(_gather_body, n_chunks),
        out_type=jax.ShapeDtypeStruct((n_idx, _D_OUT), jnp.float32),
        mesh=mesh,
        scratch_types=[
            pltpu.VMEM((n_chunks * _CHUNK,), jnp.int32),
            pltpu.VMEM((2, _CHUNK, _D_OUT), jnp.float32),
            pltpu.VMEM_SHARED((1000, _D_OUT), jnp.float32),
            pltpu.SemaphoreType.DMA((2,)),
            pltpu.SemaphoreType.DMA((2,)),
        ],
        compiler_params=pltpu.CompilerParams(use_tc_tiling_on_sc=False),
    )
    return k(table, idx_flat)


@jax.jit
def kernel(station_ids, id_emb, t_from_A, W1, b1, W2, b2):
    B, L = station_ids.shape
    table = _build_table(id_emb, t_from_A, W1, b1, W2, b2)
    out = _gather(table, station_ids.reshape(B * L))
    return out.reshape(B, L, _D_OUT)
